# baseline (speedup 1.0000x reference)
import functools

import jax
import jax.numpy as jnp
from jax.experimental import pallas as pl

N = 10000
D = 128
BN = 2000


def _combine_body(apply_relu, s_ref, sq_ref, mx_ref, mn_ref, deg_ref, avg_ref,
                  w_ref, b_ref, o_ref):
    deg = deg_ref[:, :1]
    degc = jnp.maximum(deg, 1.0)
    mean = s_ref[:] / degc
    meansq = sq_ref[:] / degc
    std = jnp.sqrt(jax.nn.relu(meansq - mean * mean) + 1e-5)
    has = deg > 0
    mx = jnp.where(has, mx_ref[:], 0.0)
    mn = jnp.where(has, mn_ref[:], 0.0)
    logd = jnp.log(deg + 1.0)
    avg = avg_ref[0, 0]
    amp = logd / avg
    att = jnp.where(has, avg / jnp.maximum(logd, 1e-5), 1.0)
    aggr = jnp.concatenate([mean, mn, mx, std], axis=1)
    w = w_ref[:]
    o = (jnp.dot(aggr, w[0:4 * D], preferred_element_type=jnp.float32)
         + amp * jnp.dot(aggr, w[4 * D:8 * D], preferred_element_type=jnp.float32)
         + att * jnp.dot(aggr, w[8 * D:12 * D], preferred_element_type=jnp.float32)
         + b_ref[:])
    if apply_relu:
        o = jax.nn.relu(o)
    o_ref[:] = o


def _combine(s, sq, mx, mn, deg2, avg_logd, W, b, apply_relu):
    grid = (N // BN,)
    row_spec = pl.BlockSpec((BN, D), lambda i: (i, 0))
    return pl.pallas_call(
        functools.partial(_combine_body, apply_relu),
        grid=grid,
        in_specs=[
            row_spec, row_spec, row_spec, row_spec,
            pl.BlockSpec((BN, 1), lambda i: (i, 0)),
            pl.BlockSpec((1, 1), lambda i: (0, 0)),
            pl.BlockSpec((12 * D, D), lambda i: (0, 0)),
            pl.BlockSpec((1, D), lambda i: (0, 0)),
        ],
        out_specs=row_spec,
        out_shape=jax.ShapeDtypeStruct((N, D), jnp.float32),
    )(s, sq, mx, mn, deg2, avg_logd, W, b)


def kernel(input_embeds, edge_index, input_index, W1, b1, W2, b2):
    src0 = edge_index[0]
    dst0 = edge_index[1]
    src = jnp.concatenate([src0, dst0])
    dst = jnp.concatenate([dst0, src0])
    x = jnp.zeros((N, D), jnp.float32).at[input_index].set(input_embeds)

    deg = jax.ops.segment_sum(jnp.ones_like(src, dtype=jnp.float32), dst,
                              num_segments=N)
    deg2 = deg[:, None]
    avg_logd = jnp.mean(jnp.log(deg + 1.0)).reshape(1, 1)
    b1r = b1.reshape(1, D)
    b2r = b2.reshape(1, D)

    def stats(x):
        msgs = jnp.take(x, src, axis=0)
        s = jax.ops.segment_sum(msgs, dst, num_segments=N)
        sq = jax.ops.segment_sum(msgs * msgs, dst, num_segments=N)
        mx = jax.ops.segment_max(msgs, dst, num_segments=N)
        mn = jax.ops.segment_min(msgs, dst, num_segments=N)
        return s, sq, mx, mn

    s, sq, mx, mn = stats(x)
    h = _combine(s, sq, mx, mn, deg2, avg_logd, W1, b1r, True)
    s, sq, mx, mn = stats(h)
    return _combine(s, sq, mx, mn, deg2, avg_logd, W2, b2r, False)
